# Initial kernel scaffold; baseline (speedup 1.0000x reference)
import jax
import jax.numpy as jnp
from jax.experimental import pallas as pl


def kernel(node_features, edge_features, edge_index, batch_index, node_W, node_b, edge_W, edge_b, msg_W1, msg_b1, msg_W2, msg_b2, upd_W1, upd_b1, upd_W2, upd_b2, ln_g, ln_b, att_W1, att_b1, att_W2, att_b2, prop_W, prop_b, feat_W, feat_b):
    raise NotImplementedError("write your pallas kernel here")


# speedup vs baseline: 3.5213x; 3.5213x over previous
import jax
import jax.numpy as jnp
from jax import lax
from jax.experimental import pallas as pl
from jax.experimental.pallas import tpu as pltpu
from jax.experimental.pallas import tpu_sc as plsc

N = 50000
E = 800000
B = 64
H = 64
NODE_IN = 128
EDGE_IN = 16

f32 = jnp.float32

_NP = N // 2
_EP = E // 2
_NBLK = 5000
_NGRID = _NP // _NBLK
_EBLK = 8000
_EGRID = _EP // _EBLK


def _proj_body(a, w, b, o):
    y = jnp.dot(a[...], w[...], preferred_element_type=f32) + b[...]
    blk = o.shape[0]
    o[...] = jnp.concatenate([y[:blk], y[blk:]], axis=1)


def _proj_nodes(nf, w, b):
    return pl.pallas_call(
        _proj_body,
        grid=(5,),
        in_specs=[
            pl.BlockSpec((10000, NODE_IN), lambda i: (i, 0)),
            pl.BlockSpec((NODE_IN, H), lambda i: (0, 0)),
            pl.BlockSpec((1, H), lambda i: (0, 0)),
        ],
        out_specs=pl.BlockSpec((_NBLK, 2 * H), lambda i: (i, 0)),
        out_shape=jax.ShapeDtypeStruct((_NP, 2 * H), f32),
        compiler_params=pltpu.CompilerParams(
            dimension_semantics=("parallel",)),
    )(nf, w, b)


def _proj_edges(ef, w, b):
    return pl.pallas_call(
        _proj_body,
        grid=(_EGRID,),
        in_specs=[
            pl.BlockSpec((2 * _EBLK, EDGE_IN), lambda i: (i, 0)),
            pl.BlockSpec((EDGE_IN, H), lambda i: (0, 0)),
            pl.BlockSpec((1, H), lambda i: (0, 0)),
        ],
        out_specs=pl.BlockSpec((_EBLK, 2 * H), lambda i: (i, 0)),
        out_shape=jax.ShapeDtypeStruct((_EP, 2 * H), f32),
        compiler_params=pltpu.CompilerParams(
            dimension_semantics=("parallel",)),
    )(ef, w, b)


def _msg_body(xs, xd, e, w1a, w1b, w1c, b1, w2, b2, o):
    t = jnp.dot(xs[...], w1a[...], preferred_element_type=f32)
    t += jnp.dot(xd[...], w1b[...], preferred_element_type=f32)
    t += jnp.dot(e[...], w1c[...], preferred_element_type=f32)
    t = jnp.maximum(t + b1[...], 0.0)
    o[...] = jnp.dot(t, w2[...], preferred_element_type=f32) + b2[...]


def _msg_mlp(xs2, xd2, e2, w1a, w1b, w1c, b1, w2, b2):
    return pl.pallas_call(
        _msg_body,
        grid=(_EGRID,),
        in_specs=[
            pl.BlockSpec((_EBLK, 2 * H), lambda i: (i, 0)),
            pl.BlockSpec((_EBLK, 2 * H), lambda i: (i, 0)),
            pl.BlockSpec((_EBLK, 2 * H), lambda i: (i, 0)),
            pl.BlockSpec((2 * H, 4 * H), lambda i: (0, 0)),
            pl.BlockSpec((2 * H, 4 * H), lambda i: (0, 0)),
            pl.BlockSpec((2 * H, 4 * H), lambda i: (0, 0)),
            pl.BlockSpec((1, 4 * H), lambda i: (0, 0)),
            pl.BlockSpec((4 * H, 2 * H), lambda i: (0, 0)),
            pl.BlockSpec((1, 2 * H), lambda i: (0, 0)),
        ],
        out_specs=pl.BlockSpec((_EBLK, 2 * H), lambda i: (i, 0)),
        out_shape=jax.ShapeDtypeStruct((_EP, 2 * H), f32),
        compiler_params=pltpu.CompilerParams(
            dimension_semantics=("parallel",)),
    )(xs2, xd2, e2, w1a, w1b, w1c, b1, w2, b2)


def _upd_body(x, agg, u1a, u1b, b1, w2, b2, g, bb, o):
    t = jnp.dot(x[...], u1a[...], preferred_element_type=f32)
    t += jnp.dot(agg[...], u1b[...], preferred_element_type=f32)
    t = jnp.maximum(t + b1[...], 0.0)
    u = jnp.dot(t, w2[...], preferred_element_type=f32) + b2[...]
    y = x[...] + u
    ya = y[:, :H]
    yb = y[:, H:]

    def ln(z):
        m = jnp.mean(z, axis=-1, keepdims=True)
        c = z - m
        v = jnp.mean(c * c, axis=-1, keepdims=True)
        return c * lax.rsqrt(v + 1e-5)

    o[...] = jnp.concatenate([ln(ya), ln(yb)], axis=-1) * g[...] + bb[...]


def _upd_mlp(x2, agg2, u1a, u1b, b1, u2, b2, g, bb):
    return pl.pallas_call(
        _upd_body,
        grid=(_NGRID,),
        in_specs=[
            pl.BlockSpec((_NBLK, 2 * H), lambda i: (i, 0)),
            pl.BlockSpec((_NBLK, 2 * H), lambda i: (i, 0)),
            pl.BlockSpec((2 * H, 2 * H), lambda i: (0, 0)),
            pl.BlockSpec((2 * H, 2 * H), lambda i: (0, 0)),
            pl.BlockSpec((1, 2 * H), lambda i: (0, 0)),
            pl.BlockSpec((2 * H, 2 * H), lambda i: (0, 0)),
            pl.BlockSpec((1, 2 * H), lambda i: (0, 0)),
            pl.BlockSpec((1, 2 * H), lambda i: (0, 0)),
            pl.BlockSpec((1, 2 * H), lambda i: (0, 0)),
        ],
        out_specs=pl.BlockSpec((_NBLK, 2 * H), lambda i: (i, 0)),
        out_shape=jax.ShapeDtypeStruct((_NP, 2 * H), f32),
        compiler_params=pltpu.CompilerParams(
            dimension_semantics=("parallel",)),
    )(x2, agg2, u1a, u1b, b1, u2, b2, g, bb)


def _att_body(x, w1, b1, w2, b2, lo, mo, acc):
    i = pl.program_id(0)

    @pl.when(i == 0)
    def _():
        acc[...] = jnp.full((1, 1), -jnp.inf, f32)

    t = jnp.maximum(jnp.dot(x[...], w1[...], preferred_element_type=f32)
                    + b1[...], 0.0)
    l = jnp.dot(t, w2[...], preferred_element_type=f32) + b2[...]
    lo[...] = l
    acc[...] = jnp.maximum(acc[...], jnp.max(l, keepdims=True))

    @pl.when(i == _NGRID - 1)
    def _():
        mo[...] = acc[...]


def _att_logits(x2, w1, b1, w2, b2):
    return pl.pallas_call(
        _att_body,
        grid=(_NGRID,),
        in_specs=[
            pl.BlockSpec((_NBLK, 2 * H), lambda i: (i, 0)),
            pl.BlockSpec((2 * H, H), lambda i: (0, 0)),
            pl.BlockSpec((1, H), lambda i: (0, 0)),
            pl.BlockSpec((H, 2), lambda i: (0, 0)),
            pl.BlockSpec((1, 2), lambda i: (0, 0)),
        ],
        out_specs=[
            pl.BlockSpec((_NBLK, 2), lambda i: (i, 0)),
            pl.BlockSpec((1, 1), lambda i: (0, 0)),
        ],
        out_shape=[
            jax.ShapeDtypeStruct((_NP, 2), f32),
            jax.ShapeDtypeStruct((1, 1), f32),
        ],
        scratch_shapes=[pltpu.VMEM((1, 1), f32)],
        compiler_params=pltpu.CompilerParams(
            dimension_semantics=("arbitrary",)),
    )(x2, w1, b1, w2, b2)


def _pool_body(x, l, bidx, gmax, pw, pb, fw, fb, pred, feat, gf_acc, w_acc):
    i = pl.program_id(0)

    @pl.when(i == 0)
    def _():
        gf_acc[...] = jnp.zeros((B, H), f32)
        w_acc[...] = jnp.zeros((1, 1), f32)

    w = jnp.exp(l[...] - gmax[...])
    seg = bidx[0]
    iota = lax.broadcasted_iota(jnp.int32, (_NBLK, B), 1)
    oh_e = (seg[:, 0:1] == iota).astype(f32)
    oh_o = (seg[:, 1:2] == iota).astype(f32)
    xe = x[:, :H] * w[:, 0:1]
    xo = x[:, H:] * w[:, 1:2]
    dn = (((0,), (0,)), ((), ()))
    gf_acc[...] += (lax.dot_general(oh_e, xe, dn, preferred_element_type=f32)
                    + lax.dot_general(oh_o, xo, dn,
                                      preferred_element_type=f32))
    w_acc[...] += jnp.sum(w, keepdims=True).reshape(1, 1)

    @pl.when(i == _NGRID - 1)
    def _():
        gf = gf_acc[...] / w_acc[...]
        pred[...] = jnp.dot(gf, pw[...], preferred_element_type=f32) + pb[...]
        feat[...] = jnp.dot(gf, fw[...], preferred_element_type=f32) + fb[...]


def _pool(x2, l2, bidx3, gmax, pw, pb, fw, fb):
    return pl.pallas_call(
        _pool_body,
        grid=(_NGRID,),
        in_specs=[
            pl.BlockSpec((_NBLK, 2 * H), lambda i: (i, 0)),
            pl.BlockSpec((_NBLK, 2), lambda i: (i, 0)),
            pl.BlockSpec((1, _NBLK, 2), lambda i: (i, 0, 0)),
            pl.BlockSpec((1, 1), lambda i: (0, 0)),
            pl.BlockSpec((H, 1), lambda i: (0, 0)),
            pl.BlockSpec((1, 1), lambda i: (0, 0)),
            pl.BlockSpec((H, H), lambda i: (0, 0)),
            pl.BlockSpec((1, H), lambda i: (0, 0)),
        ],
        out_specs=[
            pl.BlockSpec((B, 1), lambda i: (0, 0)),
            pl.BlockSpec((B, H), lambda i: (0, 0)),
        ],
        out_shape=[
            jax.ShapeDtypeStruct((B, 1), f32),
            jax.ShapeDtypeStruct((B, H), f32),
        ],
        scratch_shapes=[pltpu.VMEM((B, H), f32), pltpu.VMEM((1, 1), f32)],
        compiler_params=pltpu.CompilerParams(
            dimension_semantics=("arbitrary",)),
    )(x2, l2, bidx3, gmax, pw, pb, fw, fb)


_CW = 128
_CH = E // _CW
_NWORK = 32
_SHARD = N // 2
_PAD = _SHARD + 24
_GS = 3
_GSUP = 195 // _GS


def _gather_body(x_hbm, s_hbm, d_hbm, xs_hbm, xd_hbm,
                 idx_s, idx_d, rs, rd, sem_i, sem_g, sem_w):
    xt = x_hbm
    xsf = xs_hbm
    xdf = xd_hbm
    c = lax.axis_index("c")
    s = lax.axis_index("s")
    w = s * 2 + c
    start = 195 * w + jnp.minimum(w, 10)

    def idx_cps(j, p):
        row = start + _GS * j
        return (pltpu.make_async_copy(s_hbm.at[pl.ds(row, _GS)],
                                      idx_s.at[p], sem_i),
                pltpu.make_async_copy(d_hbm.at[pl.ds(row, _GS)],
                                      idx_d.at[p], sem_i))

    def wb_cps(j, p):
        base = (start + _GS * j) * _CW
        return (pltpu.make_async_copy(rs.at[p],
                                      xsf.at[pl.ds(base, _GS * _CW)], sem_w),
                pltpu.make_async_copy(rd.at[p],
                                      xdf.at[pl.ds(base, _GS * _CW)], sem_w))

    for cp in idx_cps(0, 0):
        cp.start()

    def body(j, _):
        p = jnp.bitwise_and(j, 1)
        for cp in idx_cps(j, p):
            cp.wait()

        @pl.when(j < _GSUP - 1)
        def _():
            for cp in idx_cps(j + 1, 1 - p):
                cp.start()

        @pl.when(j >= 2)
        def _():
            for cp in wb_cps(j - 2, p):
                cp.wait()

        gcps = []
        for q in range(_GS):
            gcps.append(pltpu.make_async_copy(
                xt.at[idx_s.at[p, q]],
                rs.at[p, pl.ds(q * _CW, _CW)], sem_g))
            gcps.append(pltpu.make_async_copy(
                xt.at[idx_d.at[p, q]],
                rd.at[p, pl.ds(q * _CW, _CW)], sem_g))
        for cp in gcps:
            cp.start()
        for cp in gcps:
            cp.wait()
        for cp in wb_cps(j, p):
            cp.start()
        return 0

    lax.fori_loop(0, _GSUP, body, 0, unroll=False)
    for cp in wb_cps(_GSUP - 2, 1):
        cp.wait()
    for cp in wb_cps(_GSUP - 1, 0):
        cp.wait()

    @pl.when(w < 10)
    def _():
        row = start + 195
        pltpu.sync_copy(s_hbm.at[row], idx_s.at[0, 0])
        pltpu.sync_copy(d_hbm.at[row], idx_d.at[0, 0])
        cp1 = pltpu.async_copy(xt.at[idx_s.at[0, 0]],
                               rs.at[0, pl.ds(0, _CW)], sem_g)
        cp2 = pltpu.async_copy(xt.at[idx_d.at[0, 0]],
                               rd.at[0, pl.ds(0, _CW)], sem_g)
        cp1.wait()
        cp2.wait()
        pltpu.sync_copy(rs.at[0, pl.ds(0, _CW)],
                        xsf.at[pl.ds(row * _CW, _CW)])
        pltpu.sync_copy(rd.at[0, pl.ds(0, _CW)],
                        xdf.at[pl.ds(row * _CW, _CW)])


def _gather(x2, src2, dst2):
    k = pl.kernel(
        _gather_body,
        out_type=(jax.ShapeDtypeStruct((E, H), f32),
                  jax.ShapeDtypeStruct((E, H), f32)),
        mesh=plsc.VectorSubcoreMesh(core_axis_name="c", subcore_axis_name="s"),
        scratch_types=[
            pltpu.VMEM((2, _GS, _CW), jnp.int32),
            pltpu.VMEM((2, _GS, _CW), jnp.int32),
            pltpu.VMEM((2, _GS * _CW, H), f32),
            pltpu.VMEM((2, _GS * _CW, H), f32),
            pltpu.SemaphoreType.DMA,
            pltpu.SemaphoreType.DMA,
            pltpu.SemaphoreType.DMA,
        ],
        compiler_params=pltpu.CompilerParams(use_tc_tiling_on_sc=False),
    )
    return k(x2, src2, dst2)


def _scatter_body(m_hbm, d_hbm, z_hbm, agg_hbm, idx_v, idx_l, m_v, shard,
                  sem_i, sem_a):
    mf = m_hbm
    zf = z_hbm
    aggf = agg_hbm
    c = lax.axis_index("c")
    s = lax.axis_index("s")
    base = c * _SHARD
    zrows = _PAD // 16
    pltpu.sync_copy(zf.at[pl.ds(s * zrows, zrows)],
                    shard.at[pl.ds(s * zrows, zrows)])
    plsc.subcore_barrier()

    start = 390 * s + jnp.minimum(s, 10)
    n = jnp.where(s < 10, 391, 390)

    def pf_cps(k, p):
        r = start + k
        return (pltpu.make_async_copy(d_hbm.at[r], idx_v.at[p], sem_i),
                pltpu.make_async_copy(mf.at[pl.ds(r * _CW, _CW)],
                                      m_v.at[p], sem_i))

    def add_cp(p):
        return pltpu.make_async_copy(m_v.at[p], shard.at[idx_l.at[p]], sem_a)

    for cp in pf_cps(0, 0):
        cp.start()

    def body(k, _):
        p = jnp.bitwise_and(k, 1)

        @pl.when(k < n)
        def _():
            for cp in pf_cps(k, p):
                cp.wait()
            for j in range(_CW // 16):
                v = idx_v[p, pl.ds(j * 16, 16)]
                inb = jnp.logical_and(v >= base, v < base + _SHARD)
                idx_l[p, pl.ds(j * 16, 16)] = jnp.where(inb, v - base,
                                                        _SHARD)
            pltpu.async_copy(m_v.at[p], shard.at[idx_l.at[p]], sem_a,
                             add=True)

            @pl.when(k >= 1)
            def _():
                add_cp(1 - p).wait()

            @pl.when(k + 1 < n)
            def _():
                for cp in pf_cps(k + 1, 1 - p):
                    cp.start()
        return 0

    lax.fori_loop(0, 391, body, 0, unroll=False)
    add_cp(jnp.bitwise_and(n - 1, 1)).wait()
    plsc.subcore_barrier()

    wrows = _SHARD // 16
    r0 = s * wrows
    pltpu.sync_copy(shard.at[pl.ds(r0, wrows)],
                    aggf.at[pl.ds(base + r0, wrows)])

    @pl.when(s == 15)
    def _():
        pltpu.sync_copy(shard.at[pl.ds(16 * wrows, _SHARD - 16 * wrows)],
                        aggf.at[pl.ds(base + 16 * wrows,
                                      _SHARD - 16 * wrows)])


def _scatter(m2, dst2, zeros2):
    k = pl.kernel(
        _scatter_body,
        out_type=jax.ShapeDtypeStruct((N, H), f32),
        mesh=plsc.VectorSubcoreMesh(core_axis_name="c", subcore_axis_name="s"),
        scratch_types=[
            pltpu.VMEM((2, _CW), jnp.int32),
            pltpu.VMEM((2, _CW), jnp.int32),
            pltpu.VMEM((2, _CW, H), f32),
            pltpu.VMEM_SHARED((_PAD, H), f32),
            pltpu.SemaphoreType.DMA,
            pltpu.SemaphoreType.DMA,
        ],
        compiler_params=pltpu.CompilerParams(use_tc_tiling_on_sc=False),
    )
    return k(m2, dst2, zeros2)


def _bd(w):
    a, b = w.shape
    z = jnp.zeros((2 * a, 2 * b), f32)
    return z.at[:a, :b].set(w).at[a:, b:].set(w)


def _db(b):
    return jnp.concatenate([b, b])[None, :]


def _phi(v):
    i = v // (2 * _NBLK)
    j = v - i * (2 * _NBLK)
    return i * (2 * _NBLK) + (j % _NBLK) * 2 + j // _NBLK


def _eperm(a):
    return a.reshape(_EGRID, 2, _EBLK).transpose(0, 2, 1).reshape(_CH, _CW)


def kernel(node_features, edge_features, edge_index, batch_index, node_W,
           node_b, edge_W, edge_b, msg_W1, msg_b1, msg_W2, msg_b2, upd_W1,
           upd_b1, upd_W2, upd_b2, ln_g, ln_b, att_W1, att_b1, att_W2,
           att_b2, prop_W, prop_b, feat_W, feat_b):
    src2 = _eperm(_phi(edge_index[0]))
    dst2 = _eperm(_phi(edge_index[1]))
    x2 = _proj_nodes(node_features, node_W, node_b.reshape(1, H))
    e2 = _proj_edges(edge_features, edge_W, edge_b.reshape(1, H))
    zeros = jnp.zeros((_PAD, H), f32)
    for l in range(3):
        xs, xd = _gather(x2.reshape(N, H), src2, dst2)
        w1 = msg_W1[l]
        m2 = _msg_mlp(xs.reshape(_EP, 2 * H), xd.reshape(_EP, 2 * H), e2,
                      _bd(w1[:H]), _bd(w1[H:2 * H]),
                      _bd(w1[2 * H:]), _db(msg_b1[l]), _bd(msg_W2[l]),
                      _db(msg_b2[l]))
        agg2 = _scatter(m2.reshape(E, H), dst2, zeros).reshape(_NP, 2 * H)
        x2 = _upd_mlp(x2, agg2, _bd(upd_W1[l][:H]), _bd(upd_W1[l][H:]),
                      _db(upd_b1[l]), _bd(upd_W2[l]), _db(upd_b2[l]),
                      _db(ln_g[l]), _db(ln_b[l]))
    l2, gmax = _att_logits(x2, _bd(att_W1), _db(att_b1), _bd(att_W2),
                           _db(att_b2))
    bidx3 = batch_index.reshape(_NGRID, 2, _NBLK).transpose(0, 2, 1)
    pred, feat = _pool(x2, l2, bidx3, gmax, prop_W, prop_b.reshape(1, 1),
                       feat_W, feat_b.reshape(1, H))
    xout = (x2.reshape(_NGRID, _NBLK, 2, H).transpose(0, 2, 1, 3)
            .reshape(N, H))
    return (pred, feat, xout)


# speedup vs baseline: 3.7365x; 1.0611x over previous
import jax
import jax.numpy as jnp
from jax import lax
from jax.experimental import pallas as pl
from jax.experimental.pallas import tpu as pltpu
from jax.experimental.pallas import tpu_sc as plsc

N = 50000
E = 800000
B = 64
H = 64
NODE_IN = 128
EDGE_IN = 16

f32 = jnp.float32

_NP = N // 2
_EP = E // 2
_NBLK = 5000
_NGRID = _NP // _NBLK
_EBLK = 8000
_EGRID = _EP // _EBLK


def _proj_body(a, w, b, o):
    y = jnp.dot(a[...], w[...], preferred_element_type=f32) + b[...]
    blk = o.shape[0]
    o[...] = jnp.concatenate([y[:blk], y[blk:]], axis=1)


def _proj_nodes(nf, w, b):
    return pl.pallas_call(
        _proj_body,
        grid=(5,),
        in_specs=[
            pl.BlockSpec((10000, NODE_IN), lambda i: (i, 0)),
            pl.BlockSpec((NODE_IN, H), lambda i: (0, 0)),
            pl.BlockSpec((1, H), lambda i: (0, 0)),
        ],
        out_specs=pl.BlockSpec((_NBLK, 2 * H), lambda i: (i, 0)),
        out_shape=jax.ShapeDtypeStruct((_NP, 2 * H), f32),
        compiler_params=pltpu.CompilerParams(
            dimension_semantics=("parallel",)),
    )(nf, w, b)


def _proj_edges_body(a, w, b, o):
    y = lax.dot_general(a[...], w[...], (((0,), (0,)), ((), ())),
                        preferred_element_type=f32) + b[...]
    blk = o.shape[0]
    o[...] = jnp.concatenate([y[:blk], y[blk:]], axis=1)


def _proj_edges(efT, w, b):
    return pl.pallas_call(
        _proj_edges_body,
        grid=(_EGRID,),
        in_specs=[
            pl.BlockSpec((EDGE_IN, 2 * _EBLK), lambda i: (0, i)),
            pl.BlockSpec((EDGE_IN, H), lambda i: (0, 0)),
            pl.BlockSpec((1, H), lambda i: (0, 0)),
        ],
        out_specs=pl.BlockSpec((_EBLK, 2 * H), lambda i: (i, 0)),
        out_shape=jax.ShapeDtypeStruct((_EP, 2 * H), f32),
        compiler_params=pltpu.CompilerParams(
            dimension_semantics=("parallel",)),
    )(efT, w, b)


def _msg_body(xs, xd, e, w1a, w1b, w1c, b1, w2, b2, o):
    t = jnp.dot(xs[...], w1a[...], preferred_element_type=f32)
    t += jnp.dot(xd[...], w1b[...], preferred_element_type=f32)
    t += jnp.dot(e[...], w1c[...], preferred_element_type=f32)
    t = jnp.maximum(t + b1[...], 0.0)
    o[...] = jnp.dot(t, w2[...], preferred_element_type=f32) + b2[...]


def _msg_mlp(xs2, xd2, e2, w1a, w1b, w1c, b1, w2, b2):
    return pl.pallas_call(
        _msg_body,
        grid=(_EGRID,),
        in_specs=[
            pl.BlockSpec((_EBLK, 2 * H), lambda i: (i, 0)),
            pl.BlockSpec((_EBLK, 2 * H), lambda i: (i, 0)),
            pl.BlockSpec((_EBLK, 2 * H), lambda i: (i, 0)),
            pl.BlockSpec((2 * H, 4 * H), lambda i: (0, 0)),
            pl.BlockSpec((2 * H, 4 * H), lambda i: (0, 0)),
            pl.BlockSpec((2 * H, 4 * H), lambda i: (0, 0)),
            pl.BlockSpec((1, 4 * H), lambda i: (0, 0)),
            pl.BlockSpec((4 * H, 2 * H), lambda i: (0, 0)),
            pl.BlockSpec((1, 2 * H), lambda i: (0, 0)),
        ],
        out_specs=pl.BlockSpec((_EBLK, 2 * H), lambda i: (i, 0)),
        out_shape=jax.ShapeDtypeStruct((_EP, 2 * H), f32),
        compiler_params=pltpu.CompilerParams(
            dimension_semantics=("parallel",)),
    )(xs2, xd2, e2, w1a, w1b, w1c, b1, w2, b2)


def _upd_body(x, agg, u1a, u1b, b1, w2, b2, g, bb, o):
    t = jnp.dot(x[...], u1a[...], preferred_element_type=f32)
    t += jnp.dot(agg[...], u1b[...], preferred_element_type=f32)
    t = jnp.maximum(t + b1[...], 0.0)
    u = jnp.dot(t, w2[...], preferred_element_type=f32) + b2[...]
    y = x[...] + u
    ya = y[:, :H]
    yb = y[:, H:]

    def ln(z):
        m = jnp.mean(z, axis=-1, keepdims=True)
        c = z - m
        v = jnp.mean(c * c, axis=-1, keepdims=True)
        return c * lax.rsqrt(v + 1e-5)

    o[...] = jnp.concatenate([ln(ya), ln(yb)], axis=-1) * g[...] + bb[...]


def _upd_mlp(x2, agg2, u1a, u1b, b1, u2, b2, g, bb):
    return pl.pallas_call(
        _upd_body,
        grid=(_NGRID,),
        in_specs=[
            pl.BlockSpec((_NBLK, 2 * H), lambda i: (i, 0)),
            pl.BlockSpec((_NBLK, 2 * H), lambda i: (i, 0)),
            pl.BlockSpec((2 * H, 2 * H), lambda i: (0, 0)),
            pl.BlockSpec((2 * H, 2 * H), lambda i: (0, 0)),
            pl.BlockSpec((1, 2 * H), lambda i: (0, 0)),
            pl.BlockSpec((2 * H, 2 * H), lambda i: (0, 0)),
            pl.BlockSpec((1, 2 * H), lambda i: (0, 0)),
            pl.BlockSpec((1, 2 * H), lambda i: (0, 0)),
            pl.BlockSpec((1, 2 * H), lambda i: (0, 0)),
        ],
        out_specs=pl.BlockSpec((_NBLK, 2 * H), lambda i: (i, 0)),
        out_shape=jax.ShapeDtypeStruct((_NP, 2 * H), f32),
        compiler_params=pltpu.CompilerParams(
            dimension_semantics=("parallel",)),
    )(x2, agg2, u1a, u1b, b1, u2, b2, g, bb)


def _att_body(x, w1, b1, w2, b2, lo, mo, acc):
    i = pl.program_id(0)

    @pl.when(i == 0)
    def _():
        acc[...] = jnp.full((1, 1), -jnp.inf, f32)

    t = jnp.maximum(jnp.dot(x[...], w1[...], preferred_element_type=f32)
                    + b1[...], 0.0)
    l = jnp.dot(t, w2[...], preferred_element_type=f32) + b2[...]
    lo[...] = l
    acc[...] = jnp.maximum(acc[...], jnp.max(l, keepdims=True))

    @pl.when(i == _NGRID - 1)
    def _():
        mo[...] = acc[...]


def _att_logits(x2, w1, b1, w2, b2):
    return pl.pallas_call(
        _att_body,
        grid=(_NGRID,),
        in_specs=[
            pl.BlockSpec((_NBLK, 2 * H), lambda i: (i, 0)),
            pl.BlockSpec((2 * H, H), lambda i: (0, 0)),
            pl.BlockSpec((1, H), lambda i: (0, 0)),
            pl.BlockSpec((H, 2), lambda i: (0, 0)),
            pl.BlockSpec((1, 2), lambda i: (0, 0)),
        ],
        out_specs=[
            pl.BlockSpec((_NBLK, 2), lambda i: (i, 0)),
            pl.BlockSpec((1, 1), lambda i: (0, 0)),
        ],
        out_shape=[
            jax.ShapeDtypeStruct((_NP, 2), f32),
            jax.ShapeDtypeStruct((1, 1), f32),
        ],
        scratch_shapes=[pltpu.VMEM((1, 1), f32)],
        compiler_params=pltpu.CompilerParams(
            dimension_semantics=("arbitrary",)),
    )(x2, w1, b1, w2, b2)


def _pool_body(x, l, bidx, gmax, pw, pb, fw, fb, pred, feat, gf_acc, w_acc):
    i = pl.program_id(0)

    @pl.when(i == 0)
    def _():
        gf_acc[...] = jnp.zeros((B, H), f32)
        w_acc[...] = jnp.zeros((1, 1), f32)

    w = jnp.exp(l[...] - gmax[...])
    seg = bidx[0]
    iota = lax.broadcasted_iota(jnp.int32, (_NBLK, B), 1)
    oh_e = (seg[:, 0:1] == iota).astype(f32)
    oh_o = (seg[:, 1:2] == iota).astype(f32)
    xe = x[:, :H] * w[:, 0:1]
    xo = x[:, H:] * w[:, 1:2]
    dn = (((0,), (0,)), ((), ()))
    gf_acc[...] += (lax.dot_general(oh_e, xe, dn, preferred_element_type=f32)
                    + lax.dot_general(oh_o, xo, dn,
                                      preferred_element_type=f32))
    w_acc[...] += jnp.sum(w, keepdims=True).reshape(1, 1)

    @pl.when(i == _NGRID - 1)
    def _():
        gf = gf_acc[...] / w_acc[...]
        pred[...] = jnp.dot(gf, pw[...], preferred_element_type=f32) + pb[...]
        feat[...] = jnp.dot(gf, fw[...], preferred_element_type=f32) + fb[...]


def _pool(x2, l2, bidx3, gmax, pw, pb, fw, fb):
    return pl.pallas_call(
        _pool_body,
        grid=(_NGRID,),
        in_specs=[
            pl.BlockSpec((_NBLK, 2 * H), lambda i: (i, 0)),
            pl.BlockSpec((_NBLK, 2), lambda i: (i, 0)),
            pl.BlockSpec((1, _NBLK, 2), lambda i: (i, 0, 0)),
            pl.BlockSpec((1, 1), lambda i: (0, 0)),
            pl.BlockSpec((H, 1), lambda i: (0, 0)),
            pl.BlockSpec((1, 1), lambda i: (0, 0)),
            pl.BlockSpec((H, H), lambda i: (0, 0)),
            pl.BlockSpec((1, H), lambda i: (0, 0)),
        ],
        out_specs=[
            pl.BlockSpec((B, 1), lambda i: (0, 0)),
            pl.BlockSpec((B, H), lambda i: (0, 0)),
        ],
        out_shape=[
            jax.ShapeDtypeStruct((B, 1), f32),
            jax.ShapeDtypeStruct((B, H), f32),
        ],
        scratch_shapes=[pltpu.VMEM((B, H), f32), pltpu.VMEM((1, 1), f32)],
        compiler_params=pltpu.CompilerParams(
            dimension_semantics=("arbitrary",)),
    )(x2, l2, bidx3, gmax, pw, pb, fw, fb)


_CW = 128
_CH = E // _CW
_NWORK = 32
_SHARD = N // 2
_PAD = _SHARD + 24
_GS = 3
_GSUP = 195 // _GS


def _gather_body(x_hbm, s_hbm, d_hbm, xs_hbm, xd_hbm,
                 idx_s, idx_d, rs, rd, sem_i, sem_g, sem_w):
    xt = x_hbm
    xsf = xs_hbm
    xdf = xd_hbm
    c = lax.axis_index("c")
    s = lax.axis_index("s")
    w = s * 2 + c
    start = 195 * w + jnp.minimum(w, 10)

    def idx_cps(j, p):
        row = start + _GS * j
        return (pltpu.make_async_copy(s_hbm.at[pl.ds(row, _GS)],
                                      idx_s.at[p], sem_i),
                pltpu.make_async_copy(d_hbm.at[pl.ds(row, _GS)],
                                      idx_d.at[p], sem_i))

    def wb_cps(j, p):
        base = (start + _GS * j) * _CW
        return (pltpu.make_async_copy(rs.at[p],
                                      xsf.at[pl.ds(base, _GS * _CW)], sem_w),
                pltpu.make_async_copy(rd.at[p],
                                      xdf.at[pl.ds(base, _GS * _CW)], sem_w))

    for cp in idx_cps(0, 0):
        cp.start()

    def body(j, _):
        p = jnp.bitwise_and(j, 1)
        for cp in idx_cps(j, p):
            cp.wait()

        @pl.when(j < _GSUP - 1)
        def _():
            for cp in idx_cps(j + 1, 1 - p):
                cp.start()

        @pl.when(j >= 2)
        def _():
            for cp in wb_cps(j - 2, p):
                cp.wait()

        gcps = []
        for q in range(_GS):
            gcps.append(pltpu.make_async_copy(
                xt.at[idx_s.at[p, q]],
                rs.at[p, pl.ds(q * _CW, _CW)], sem_g))
            gcps.append(pltpu.make_async_copy(
                xt.at[idx_d.at[p, q]],
                rd.at[p, pl.ds(q * _CW, _CW)], sem_g))
        for cp in gcps:
            cp.start()
        for cp in gcps:
            cp.wait()
        for cp in wb_cps(j, p):
            cp.start()
        return 0

    lax.fori_loop(0, _GSUP, body, 0, unroll=False)
    for cp in wb_cps(_GSUP - 2, 1):
        cp.wait()
    for cp in wb_cps(_GSUP - 1, 0):
        cp.wait()

    @pl.when(w < 10)
    def _():
        row = start + 195
        pltpu.sync_copy(s_hbm.at[row], idx_s.at[0, 0])
        pltpu.sync_copy(d_hbm.at[row], idx_d.at[0, 0])
        cp1 = pltpu.async_copy(xt.at[idx_s.at[0, 0]],
                               rs.at[0, pl.ds(0, _CW)], sem_g)
        cp2 = pltpu.async_copy(xt.at[idx_d.at[0, 0]],
                               rd.at[0, pl.ds(0, _CW)], sem_g)
        cp1.wait()
        cp2.wait()
        pltpu.sync_copy(rs.at[0, pl.ds(0, _CW)],
                        xsf.at[pl.ds(row * _CW, _CW)])
        pltpu.sync_copy(rd.at[0, pl.ds(0, _CW)],
                        xdf.at[pl.ds(row * _CW, _CW)])


def _gather(x2, src2, dst2):
    k = pl.kernel(
        _gather_body,
        out_type=(jax.ShapeDtypeStruct((E, H), f32),
                  jax.ShapeDtypeStruct((E, H), f32)),
        mesh=plsc.VectorSubcoreMesh(core_axis_name="c", subcore_axis_name="s"),
        scratch_types=[
            pltpu.VMEM((2, _GS, _CW), jnp.int32),
            pltpu.VMEM((2, _GS, _CW), jnp.int32),
            pltpu.VMEM((2, _GS * _CW, H), f32),
            pltpu.VMEM((2, _GS * _CW, H), f32),
            pltpu.SemaphoreType.DMA,
            pltpu.SemaphoreType.DMA,
            pltpu.SemaphoreType.DMA,
        ],
        compiler_params=pltpu.CompilerParams(use_tc_tiling_on_sc=False),
    )
    return k(x2, src2, dst2)


def _scatter_body(m_hbm, d_hbm, z_hbm, agg_hbm, idx_v, idx_l, m_v, shard,
                  sem_i, sem_a):
    mf = m_hbm
    zf = z_hbm
    aggf = agg_hbm
    c = lax.axis_index("c")
    s = lax.axis_index("s")
    base = c * _SHARD
    zrows = _PAD // 16
    pltpu.sync_copy(zf.at[pl.ds(s * zrows, zrows)],
                    shard.at[pl.ds(s * zrows, zrows)])
    plsc.subcore_barrier()

    start = 390 * s + jnp.minimum(s, 10)
    n = jnp.where(s < 10, 391, 390)

    def pf_cps(k, p):
        r = start + k
        return (pltpu.make_async_copy(d_hbm.at[r], idx_v.at[p], sem_i),
                pltpu.make_async_copy(mf.at[pl.ds(r * _CW, _CW)],
                                      m_v.at[p], sem_i))

    def add_cp(p):
        return pltpu.make_async_copy(m_v.at[p], shard.at[idx_l.at[p]], sem_a)

    for cp in pf_cps(0, 0):
        cp.start()

    def body(k, _):
        p = jnp.bitwise_and(k, 1)

        @pl.when(k < n)
        def _():
            for cp in pf_cps(k, p):
                cp.wait()
            for j in range(_CW // 16):
                v = idx_v[p, pl.ds(j * 16, 16)]
                inb = jnp.logical_and(v >= base, v < base + _SHARD)
                idx_l[p, pl.ds(j * 16, 16)] = jnp.where(inb, v - base,
                                                        _SHARD)
            pltpu.async_copy(m_v.at[p], shard.at[idx_l.at[p]], sem_a,
                             add=True)

            @pl.when(k >= 1)
            def _():
                add_cp(1 - p).wait()

            @pl.when(k + 1 < n)
            def _():
                for cp in pf_cps(k + 1, 1 - p):
                    cp.start()
        return 0

    lax.fori_loop(0, 391, body, 0, unroll=False)
    add_cp(jnp.bitwise_and(n - 1, 1)).wait()
    plsc.subcore_barrier()

    wrows = _SHARD // 16
    r0 = s * wrows
    pltpu.sync_copy(shard.at[pl.ds(r0, wrows)],
                    aggf.at[pl.ds(base + r0, wrows)])

    @pl.when(s == 15)
    def _():
        pltpu.sync_copy(shard.at[pl.ds(16 * wrows, _SHARD - 16 * wrows)],
                        aggf.at[pl.ds(base + 16 * wrows,
                                      _SHARD - 16 * wrows)])


def _scatter(m2, dst2, zeros2):
    k = pl.kernel(
        _scatter_body,
        out_type=jax.ShapeDtypeStruct((N, H), f32),
        mesh=plsc.VectorSubcoreMesh(core_axis_name="c", subcore_axis_name="s"),
        scratch_types=[
            pltpu.VMEM((2, _CW), jnp.int32),
            pltpu.VMEM((2, _CW), jnp.int32),
            pltpu.VMEM((2, _CW, H), f32),
            pltpu.VMEM_SHARED((_PAD, H), f32),
            pltpu.SemaphoreType.DMA,
            pltpu.SemaphoreType.DMA,
        ],
        compiler_params=pltpu.CompilerParams(use_tc_tiling_on_sc=False),
    )
    return k(m2, dst2, zeros2)


def _bd(w):
    a, b = w.shape
    z = jnp.zeros((2 * a, 2 * b), f32)
    return z.at[:a, :b].set(w).at[a:, b:].set(w)


def _db(b):
    return jnp.concatenate([b, b])[None, :]


def _phi(v):
    i = v // (2 * _NBLK)
    j = v - i * (2 * _NBLK)
    return i * (2 * _NBLK) + (j % _NBLK) * 2 + j // _NBLK


def _eperm(a):
    return a.reshape(_EGRID, 2, _EBLK).transpose(0, 2, 1).reshape(_CH, _CW)


def kernel(node_features, edge_features, edge_index, batch_index, node_W,
           node_b, edge_W, edge_b, msg_W1, msg_b1, msg_W2, msg_b2, upd_W1,
           upd_b1, upd_W2, upd_b2, ln_g, ln_b, att_W1, att_b1, att_W2,
           att_b2, prop_W, prop_b, feat_W, feat_b):
    src2 = _eperm(_phi(edge_index[0]))
    dst2 = _eperm(_phi(edge_index[1]))
    x2 = _proj_nodes(node_features, node_W, node_b.reshape(1, H))
    e2 = _proj_edges(edge_features.T, edge_W, edge_b.reshape(1, H))
    zeros = jnp.zeros((_PAD, H), f32)
    for l in range(3):
        xs, xd = _gather(x2.reshape(N, H), src2, dst2)
        w1 = msg_W1[l]
        m2 = _msg_mlp(xs.reshape(_EP, 2 * H), xd.reshape(_EP, 2 * H), e2,
                      _bd(w1[:H]), _bd(w1[H:2 * H]),
                      _bd(w1[2 * H:]), _db(msg_b1[l]), _bd(msg_W2[l]),
                      _db(msg_b2[l]))
        agg2 = _scatter(m2.reshape(E, H), dst2, zeros).reshape(_NP, 2 * H)
        x2 = _upd_mlp(x2, agg2, _bd(upd_W1[l][:H]), _bd(upd_W1[l][H:]),
                      _db(upd_b1[l]), _bd(upd_W2[l]), _db(upd_b2[l]),
                      _db(ln_g[l]), _db(ln_b[l]))
    l2, gmax = _att_logits(x2, _bd(att_W1), _db(att_b1), _bd(att_W2),
                           _db(att_b2))
    bidx3 = batch_index.reshape(_NGRID, 2, _NBLK).transpose(0, 2, 1)
    pred, feat = _pool(x2, l2, bidx3, gmax, prop_W, prop_b.reshape(1, 1),
                       feat_W, feat_b.reshape(1, H))
    xout = (x2.reshape(_NGRID, _NBLK, 2, H).transpose(0, 2, 1, 3)
            .reshape(N, H))
    return (pred, feat, xout)


# speedup vs baseline: 3.8803x; 1.0385x over previous
import jax
import jax.numpy as jnp
from jax import lax
from jax.experimental import pallas as pl
from jax.experimental.pallas import tpu as pltpu
from jax.experimental.pallas import tpu_sc as plsc

N = 50000
E = 800000
B = 64
H = 64
NODE_IN = 128
EDGE_IN = 16

f32 = jnp.float32

_NP = N // 2
_EP = E // 2
_NBLK = 5000
_NGRID = _NP // _NBLK
_EBLK = 8000
_EGRID = _EP // _EBLK


def _proj_body(a, w, b, o):
    y = jnp.dot(a[...], w[...], preferred_element_type=f32) + b[...]
    blk = o.shape[0]
    o[...] = jnp.concatenate([y[:blk], y[blk:]], axis=1)


def _proj_nodes(nf, w, b):
    return pl.pallas_call(
        _proj_body,
        grid=(5,),
        in_specs=[
            pl.BlockSpec((10000, NODE_IN), lambda i: (i, 0)),
            pl.BlockSpec((NODE_IN, H), lambda i: (0, 0)),
            pl.BlockSpec((1, H), lambda i: (0, 0)),
        ],
        out_specs=pl.BlockSpec((_NBLK, 2 * H), lambda i: (i, 0)),
        out_shape=jax.ShapeDtypeStruct((_NP, 2 * H), f32),
        compiler_params=pltpu.CompilerParams(
            dimension_semantics=("parallel",)),
    )(nf, w, b)


def _proj_edges_body(a, w, b, o):
    y = lax.dot_general(a[...], w[...], (((0,), (0,)), ((), ())),
                        preferred_element_type=f32) + b[...]
    blk = o.shape[0]
    o[...] = jnp.concatenate([y[:blk], y[blk:]], axis=1)


def _proj_edges(efT, w, b):
    return pl.pallas_call(
        _proj_edges_body,
        grid=(_EGRID,),
        in_specs=[
            pl.BlockSpec((EDGE_IN, 2 * _EBLK), lambda i: (0, i)),
            pl.BlockSpec((EDGE_IN, H), lambda i: (0, 0)),
            pl.BlockSpec((1, H), lambda i: (0, 0)),
        ],
        out_specs=pl.BlockSpec((_EBLK, 2 * H), lambda i: (i, 0)),
        out_shape=jax.ShapeDtypeStruct((_EP, 2 * H), f32),
        compiler_params=pltpu.CompilerParams(
            dimension_semantics=("parallel",)),
    )(efT, w, b)


def _msg_body(xs, xd, efT, w1a, w1b, wec, b1, w2, b2, o):
    t = jnp.dot(xs[...], w1a[...], preferred_element_type=f32)
    t += jnp.dot(xd[...], w1b[...], preferred_element_type=f32)
    ec = lax.dot_general(efT[...], wec[...], (((0,), (0,)), ((), ())),
                         preferred_element_type=f32)
    t += jnp.concatenate([ec[:_EBLK], ec[_EBLK:]], axis=1)
    t = jnp.maximum(t + b1[...], 0.0)
    o[...] = jnp.dot(t, w2[...], preferred_element_type=f32) + b2[...]


def _msg_mlp(xs2, xd2, efT, w1a, w1b, wec, b1, w2, b2):
    return pl.pallas_call(
        _msg_body,
        grid=(_EGRID,),
        in_specs=[
            pl.BlockSpec((_EBLK, 2 * H), lambda i: (i, 0)),
            pl.BlockSpec((_EBLK, 2 * H), lambda i: (i, 0)),
            pl.BlockSpec((EDGE_IN, 2 * _EBLK), lambda i: (0, i)),
            pl.BlockSpec((2 * H, 4 * H), lambda i: (0, 0)),
            pl.BlockSpec((2 * H, 4 * H), lambda i: (0, 0)),
            pl.BlockSpec((EDGE_IN, 2 * H), lambda i: (0, 0)),
            pl.BlockSpec((1, 4 * H), lambda i: (0, 0)),
            pl.BlockSpec((4 * H, 2 * H), lambda i: (0, 0)),
            pl.BlockSpec((1, 2 * H), lambda i: (0, 0)),
        ],
        out_specs=pl.BlockSpec((_EBLK, 2 * H), lambda i: (i, 0)),
        out_shape=jax.ShapeDtypeStruct((_EP, 2 * H), f32),
        compiler_params=pltpu.CompilerParams(
            dimension_semantics=("parallel",)),
    )(xs2, xd2, efT, w1a, w1b, wec, b1, w2, b2)


def _upd_body(x, agg, u1a, u1b, b1, w2, b2, g, bb, o):
    t = jnp.dot(x[...], u1a[...], preferred_element_type=f32)
    t += jnp.dot(agg[...], u1b[...], preferred_element_type=f32)
    t = jnp.maximum(t + b1[...], 0.0)
    u = jnp.dot(t, w2[...], preferred_element_type=f32) + b2[...]
    y = x[...] + u
    ya = y[:, :H]
    yb = y[:, H:]

    def ln(z):
        m = jnp.mean(z, axis=-1, keepdims=True)
        c = z - m
        v = jnp.mean(c * c, axis=-1, keepdims=True)
        return c * lax.rsqrt(v + 1e-5)

    o[...] = jnp.concatenate([ln(ya), ln(yb)], axis=-1) * g[...] + bb[...]


def _upd_mlp(x2, agg2, u1a, u1b, b1, u2, b2, g, bb):
    return pl.pallas_call(
        _upd_body,
        grid=(_NGRID,),
        in_specs=[
            pl.BlockSpec((_NBLK, 2 * H), lambda i: (i, 0)),
            pl.BlockSpec((_NBLK, 2 * H), lambda i: (i, 0)),
            pl.BlockSpec((2 * H, 2 * H), lambda i: (0, 0)),
            pl.BlockSpec((2 * H, 2 * H), lambda i: (0, 0)),
            pl.BlockSpec((1, 2 * H), lambda i: (0, 0)),
            pl.BlockSpec((2 * H, 2 * H), lambda i: (0, 0)),
            pl.BlockSpec((1, 2 * H), lambda i: (0, 0)),
            pl.BlockSpec((1, 2 * H), lambda i: (0, 0)),
            pl.BlockSpec((1, 2 * H), lambda i: (0, 0)),
        ],
        out_specs=pl.BlockSpec((_NBLK, 2 * H), lambda i: (i, 0)),
        out_shape=jax.ShapeDtypeStruct((_NP, 2 * H), f32),
        compiler_params=pltpu.CompilerParams(
            dimension_semantics=("parallel",)),
    )(x2, agg2, u1a, u1b, b1, u2, b2, g, bb)


def _att_body(x, w1, b1, w2, b2, lo, mo, acc):
    i = pl.program_id(0)

    @pl.when(i == 0)
    def _():
        acc[...] = jnp.full((1, 1), -jnp.inf, f32)

    t = jnp.maximum(jnp.dot(x[...], w1[...], preferred_element_type=f32)
                    + b1[...], 0.0)
    l = jnp.dot(t, w2[...], preferred_element_type=f32) + b2[...]
    lo[...] = l
    acc[...] = jnp.maximum(acc[...], jnp.max(l, keepdims=True))

    @pl.when(i == _NGRID - 1)
    def _():
        mo[...] = acc[...]


def _att_logits(x2, w1, b1, w2, b2):
    return pl.pallas_call(
        _att_body,
        grid=(_NGRID,),
        in_specs=[
            pl.BlockSpec((_NBLK, 2 * H), lambda i: (i, 0)),
            pl.BlockSpec((2 * H, H), lambda i: (0, 0)),
            pl.BlockSpec((1, H), lambda i: (0, 0)),
            pl.BlockSpec((H, 2), lambda i: (0, 0)),
            pl.BlockSpec((1, 2), lambda i: (0, 0)),
        ],
        out_specs=[
            pl.BlockSpec((_NBLK, 2), lambda i: (i, 0)),
            pl.BlockSpec((1, 1), lambda i: (0, 0)),
        ],
        out_shape=[
            jax.ShapeDtypeStruct((_NP, 2), f32),
            jax.ShapeDtypeStruct((1, 1), f32),
        ],
        scratch_shapes=[pltpu.VMEM((1, 1), f32)],
        compiler_params=pltpu.CompilerParams(
            dimension_semantics=("arbitrary",)),
    )(x2, w1, b1, w2, b2)


def _pool_body(x, l, bidx, gmax, pw, pb, fw, fb, pred, feat, gf_acc, w_acc):
    i = pl.program_id(0)

    @pl.when(i == 0)
    def _():
        gf_acc[...] = jnp.zeros((B, H), f32)
        w_acc[...] = jnp.zeros((1, 1), f32)

    w = jnp.exp(l[...] - gmax[...])
    seg = bidx[0]
    iota = lax.broadcasted_iota(jnp.int32, (_NBLK, B), 1)
    oh_e = (seg[:, 0:1] == iota).astype(f32)
    oh_o = (seg[:, 1:2] == iota).astype(f32)
    xe = x[:, :H] * w[:, 0:1]
    xo = x[:, H:] * w[:, 1:2]
    dn = (((0,), (0,)), ((), ()))
    gf_acc[...] += (lax.dot_general(oh_e, xe, dn, preferred_element_type=f32)
                    + lax.dot_general(oh_o, xo, dn,
                                      preferred_element_type=f32))
    w_acc[...] += jnp.sum(w, keepdims=True).reshape(1, 1)

    @pl.when(i == _NGRID - 1)
    def _():
        gf = gf_acc[...] / w_acc[...]
        pred[...] = jnp.dot(gf, pw[...], preferred_element_type=f32) + pb[...]
        feat[...] = jnp.dot(gf, fw[...], preferred_element_type=f32) + fb[...]


def _pool(x2, l2, bidx3, gmax, pw, pb, fw, fb):
    return pl.pallas_call(
        _pool_body,
        grid=(_NGRID,),
        in_specs=[
            pl.BlockSpec((_NBLK, 2 * H), lambda i: (i, 0)),
            pl.BlockSpec((_NBLK, 2), lambda i: (i, 0)),
            pl.BlockSpec((1, _NBLK, 2), lambda i: (i, 0, 0)),
            pl.BlockSpec((1, 1), lambda i: (0, 0)),
            pl.BlockSpec((H, 1), lambda i: (0, 0)),
            pl.BlockSpec((1, 1), lambda i: (0, 0)),
            pl.BlockSpec((H, H), lambda i: (0, 0)),
            pl.BlockSpec((1, H), lambda i: (0, 0)),
        ],
        out_specs=[
            pl.BlockSpec((B, 1), lambda i: (0, 0)),
            pl.BlockSpec((B, H), lambda i: (0, 0)),
        ],
        out_shape=[
            jax.ShapeDtypeStruct((B, 1), f32),
            jax.ShapeDtypeStruct((B, H), f32),
        ],
        scratch_shapes=[pltpu.VMEM((B, H), f32), pltpu.VMEM((1, 1), f32)],
        compiler_params=pltpu.CompilerParams(
            dimension_semantics=("arbitrary",)),
    )(x2, l2, bidx3, gmax, pw, pb, fw, fb)


_CW = 128
_CH = E // _CW
_NWORK = 32
_SHARD = N // 2
_PAD = _SHARD + 24
_GS = 3
_GSUP = 195 // _GS


def _gather_body(x_hbm, s_hbm, d_hbm, xs_hbm, xd_hbm,
                 idx_s, idx_d, rs, rd, sem_i, sem_g, sem_w):
    xt = x_hbm
    xsf = xs_hbm
    xdf = xd_hbm
    c = lax.axis_index("c")
    s = lax.axis_index("s")
    w = s * 2 + c
    start = 195 * w + jnp.minimum(w, 10)

    def idx_cps(j, p):
        row = start + _GS * j
        return (pltpu.make_async_copy(s_hbm.at[pl.ds(row, _GS)],
                                      idx_s.at[p], sem_i),
                pltpu.make_async_copy(d_hbm.at[pl.ds(row, _GS)],
                                      idx_d.at[p], sem_i))

    def wb_cps(j, p):
        base = (start + _GS * j) * _CW
        return (pltpu.make_async_copy(rs.at[p],
                                      xsf.at[pl.ds(base, _GS * _CW)], sem_w),
                pltpu.make_async_copy(rd.at[p],
                                      xdf.at[pl.ds(base, _GS * _CW)], sem_w))

    for cp in idx_cps(0, 0):
        cp.start()

    def body(j, _):
        p = jnp.bitwise_and(j, 1)
        for cp in idx_cps(j, p):
            cp.wait()

        @pl.when(j < _GSUP - 1)
        def _():
            for cp in idx_cps(j + 1, 1 - p):
                cp.start()

        @pl.when(j >= 2)
        def _():
            for cp in wb_cps(j - 2, p):
                cp.wait()

        gcps = []
        for q in range(_GS):
            gcps.append(pltpu.make_async_copy(
                xt.at[idx_s.at[p, q]],
                rs.at[p, pl.ds(q * _CW, _CW)], sem_g))
            gcps.append(pltpu.make_async_copy(
                xt.at[idx_d.at[p, q]],
                rd.at[p, pl.ds(q * _CW, _CW)], sem_g))
        for cp in gcps:
            cp.start()
        for cp in gcps:
            cp.wait()
        for cp in wb_cps(j, p):
            cp.start()
        return 0

    lax.fori_loop(0, _GSUP, body, 0, unroll=False)
    for cp in wb_cps(_GSUP - 2, 1):
        cp.wait()
    for cp in wb_cps(_GSUP - 1, 0):
        cp.wait()

    @pl.when(w < 10)
    def _():
        row = start + 195
        pltpu.sync_copy(s_hbm.at[row], idx_s.at[0, 0])
        pltpu.sync_copy(d_hbm.at[row], idx_d.at[0, 0])
        cp1 = pltpu.async_copy(xt.at[idx_s.at[0, 0]],
                               rs.at[0, pl.ds(0, _CW)], sem_g)
        cp2 = pltpu.async_copy(xt.at[idx_d.at[0, 0]],
                               rd.at[0, pl.ds(0, _CW)], sem_g)
        cp1.wait()
        cp2.wait()
        pltpu.sync_copy(rs.at[0, pl.ds(0, _CW)],
                        xsf.at[pl.ds(row * _CW, _CW)])
        pltpu.sync_copy(rd.at[0, pl.ds(0, _CW)],
                        xdf.at[pl.ds(row * _CW, _CW)])


def _gather(x2, src2, dst2):
    k = pl.kernel(
        _gather_body,
        out_type=(jax.ShapeDtypeStruct((E, H), f32),
                  jax.ShapeDtypeStruct((E, H), f32)),
        mesh=plsc.VectorSubcoreMesh(core_axis_name="c", subcore_axis_name="s"),
        scratch_types=[
            pltpu.VMEM((2, _GS, _CW), jnp.int32),
            pltpu.VMEM((2, _GS, _CW), jnp.int32),
            pltpu.VMEM((2, _GS * _CW, H), f32),
            pltpu.VMEM((2, _GS * _CW, H), f32),
            pltpu.SemaphoreType.DMA,
            pltpu.SemaphoreType.DMA,
            pltpu.SemaphoreType.DMA,
        ],
        compiler_params=pltpu.CompilerParams(use_tc_tiling_on_sc=False),
    )
    return k(x2, src2, dst2)


def _scatter_body(m_hbm, d_hbm, z_hbm, agg_hbm, idx_v, idx_l, m_v, shard,
                  sem_i, sem_a):
    mf = m_hbm
    zf = z_hbm
    aggf = agg_hbm
    c = lax.axis_index("c")
    s = lax.axis_index("s")
    base = c * _SHARD
    zrows = _PAD // 16
    pltpu.sync_copy(zf.at[pl.ds(s * zrows, zrows)],
                    shard.at[pl.ds(s * zrows, zrows)])
    plsc.subcore_barrier()

    start = 390 * s + jnp.minimum(s, 10)
    n = jnp.where(s < 10, 391, 390)

    def pf_cps(k, p):
        r = start + k
        return (pltpu.make_async_copy(d_hbm.at[r], idx_v.at[p], sem_i),
                pltpu.make_async_copy(mf.at[pl.ds(r * _CW, _CW)],
                                      m_v.at[p], sem_i))

    def add_cp(p):
        return pltpu.make_async_copy(m_v.at[p], shard.at[idx_l.at[p]], sem_a)

    for cp in pf_cps(0, 0):
        cp.start()

    def body(k, _):
        p = jnp.bitwise_and(k, 1)

        @pl.when(k < n)
        def _():
            for cp in pf_cps(k, p):
                cp.wait()
            for j in range(_CW // 16):
                v = idx_v[p, pl.ds(j * 16, 16)]
                inb = jnp.logical_and(v >= base, v < base + _SHARD)
                idx_l[p, pl.ds(j * 16, 16)] = jnp.where(inb, v - base,
                                                        _SHARD)
            pltpu.async_copy(m_v.at[p], shard.at[idx_l.at[p]], sem_a,
                             add=True)

            @pl.when(k >= 1)
            def _():
                add_cp(1 - p).wait()

            @pl.when(k + 1 < n)
            def _():
                for cp in pf_cps(k + 1, 1 - p):
                    cp.start()
        return 0

    lax.fori_loop(0, 391, body, 0, unroll=False)
    add_cp(jnp.bitwise_and(n - 1, 1)).wait()
    plsc.subcore_barrier()

    wrows = _SHARD // 16
    r0 = s * wrows
    pltpu.sync_copy(shard.at[pl.ds(r0, wrows)],
                    aggf.at[pl.ds(base + r0, wrows)])

    @pl.when(s == 15)
    def _():
        pltpu.sync_copy(shard.at[pl.ds(16 * wrows, _SHARD - 16 * wrows)],
                        aggf.at[pl.ds(base + 16 * wrows,
                                      _SHARD - 16 * wrows)])


def _scatter(m2, dst2, zeros2):
    k = pl.kernel(
        _scatter_body,
        out_type=jax.ShapeDtypeStruct((N, H), f32),
        mesh=plsc.VectorSubcoreMesh(core_axis_name="c", subcore_axis_name="s"),
        scratch_types=[
            pltpu.VMEM((2, _CW), jnp.int32),
            pltpu.VMEM((2, _CW), jnp.int32),
            pltpu.VMEM((2, _CW, H), f32),
            pltpu.VMEM_SHARED((_PAD, H), f32),
            pltpu.SemaphoreType.DMA,
            pltpu.SemaphoreType.DMA,
        ],
        compiler_params=pltpu.CompilerParams(use_tc_tiling_on_sc=False),
    )
    return k(m2, dst2, zeros2)


def _bd(w):
    a, b = w.shape
    z = jnp.zeros((2 * a, 2 * b), f32)
    return z.at[:a, :b].set(w).at[a:, b:].set(w)


def _db(b):
    return jnp.concatenate([b, b])[None, :]


def _phi(v):
    i = v // (2 * _NBLK)
    j = v - i * (2 * _NBLK)
    return i * (2 * _NBLK) + (j % _NBLK) * 2 + j // _NBLK


def _eperm(a):
    return a.reshape(_EGRID, 2, _EBLK).transpose(0, 2, 1).reshape(_CH, _CW)


def kernel(node_features, edge_features, edge_index, batch_index, node_W,
           node_b, edge_W, edge_b, msg_W1, msg_b1, msg_W2, msg_b2, upd_W1,
           upd_b1, upd_W2, upd_b2, ln_g, ln_b, att_W1, att_b1, att_W2,
           att_b2, prop_W, prop_b, feat_W, feat_b):
    src2 = _eperm(_phi(edge_index[0]))
    dst2 = _eperm(_phi(edge_index[1]))
    x2 = _proj_nodes(node_features, node_W, node_b.reshape(1, H))
    efT = edge_features.T
    zeros = jnp.zeros((_PAD, H), f32)
    for l in range(3):
        xs, xd = _gather(x2.reshape(N, H), src2, dst2)
        w1 = msg_W1[l]
        wec = edge_W @ w1[2 * H:]
        bec = edge_b @ w1[2 * H:] + msg_b1[l]
        m2 = _msg_mlp(xs.reshape(_EP, 2 * H), xd.reshape(_EP, 2 * H), efT,
                      _bd(w1[:H]), _bd(w1[H:2 * H]),
                      wec, _db(bec), _bd(msg_W2[l]),
                      _db(msg_b2[l]))
        agg2 = _scatter(m2.reshape(E, H), dst2, zeros).reshape(_NP, 2 * H)
        x2 = _upd_mlp(x2, agg2, _bd(upd_W1[l][:H]), _bd(upd_W1[l][H:]),
                      _db(upd_b1[l]), _bd(upd_W2[l]), _db(upd_b2[l]),
                      _db(ln_g[l]), _db(ln_b[l]))
    l2, gmax = _att_logits(x2, _bd(att_W1), _db(att_b1), _bd(att_W2),
                           _db(att_b2))
    bidx3 = batch_index.reshape(_NGRID, 2, _NBLK).transpose(0, 2, 1)
    pred, feat = _pool(x2, l2, bidx3, gmax, prop_W, prop_b.reshape(1, 1),
                       feat_W, feat_b.reshape(1, H))
    xout = (x2.reshape(_NGRID, _NBLK, 2, H).transpose(0, 2, 1, 3)
            .reshape(N, H))
    return (pred, feat, xout)


# speedup vs baseline: 3.8812x; 1.0002x over previous
import jax
import jax.numpy as jnp
from jax import lax
from jax.experimental import pallas as pl
from jax.experimental.pallas import tpu as pltpu
from jax.experimental.pallas import tpu_sc as plsc

N = 50000
E = 800000
B = 64
H = 64
NODE_IN = 128
EDGE_IN = 16

f32 = jnp.float32

_NP = N // 2
_EP = E // 2
_NBLK = 5000
_NGRID = _NP // _NBLK
_EBLK = 8000
_EGRID = _EP // _EBLK


def _proj_body(a, w, b, o):
    y = jnp.dot(a[...], w[...], preferred_element_type=f32) + b[...]
    blk = o.shape[0]
    o[...] = jnp.concatenate([y[:blk], y[blk:]], axis=1)


def _proj_nodes(nf, w, b):
    return pl.pallas_call(
        _proj_body,
        grid=(5,),
        in_specs=[
            pl.BlockSpec((10000, NODE_IN), lambda i: (i, 0)),
            pl.BlockSpec((NODE_IN, H), lambda i: (0, 0)),
            pl.BlockSpec((1, H), lambda i: (0, 0)),
        ],
        out_specs=pl.BlockSpec((_NBLK, 2 * H), lambda i: (i, 0)),
        out_shape=jax.ShapeDtypeStruct((_NP, 2 * H), f32),
        compiler_params=pltpu.CompilerParams(
            dimension_semantics=("parallel",)),
    )(nf, w, b)


def _proj_edges_body(a, w, b, o):
    y = lax.dot_general(a[...], w[...], (((0,), (0,)), ((), ())),
                        preferred_element_type=f32) + b[...]
    blk = o.shape[0]
    o[...] = jnp.concatenate([y[:blk], y[blk:]], axis=1)


def _proj_edges(efT, w, b):
    return pl.pallas_call(
        _proj_edges_body,
        grid=(_EGRID,),
        in_specs=[
            pl.BlockSpec((EDGE_IN, 2 * _EBLK), lambda i: (0, i)),
            pl.BlockSpec((EDGE_IN, H), lambda i: (0, 0)),
            pl.BlockSpec((1, H), lambda i: (0, 0)),
        ],
        out_specs=pl.BlockSpec((_EBLK, 2 * H), lambda i: (i, 0)),
        out_shape=jax.ShapeDtypeStruct((_EP, 2 * H), f32),
        compiler_params=pltpu.CompilerParams(
            dimension_semantics=("parallel",)),
    )(efT, w, b)


def _msg_body(xs, xd, efT, w1a, w1b, wec, b1, w2, b2, o):
    t = jnp.dot(xs[...], w1a[...], preferred_element_type=f32)
    t += jnp.dot(xd[...], w1b[...], preferred_element_type=f32)
    ec = lax.dot_general(efT[...], wec[...], (((0,), (0,)), ((), ())),
                         preferred_element_type=f32)
    t += jnp.concatenate([ec[:_EBLK], ec[_EBLK:]], axis=1)
    t = jnp.maximum(t + b1[...], 0.0)
    o[...] = jnp.dot(t, w2[...], preferred_element_type=f32) + b2[...]


def _msg_mlp(xs2, xd2, efT, w1a, w1b, wec, b1, w2, b2):
    return pl.pallas_call(
        _msg_body,
        grid=(_EGRID,),
        in_specs=[
            pl.BlockSpec((_EBLK, 2 * H), lambda i: (i, 0)),
            pl.BlockSpec((_EBLK, 2 * H), lambda i: (i, 0)),
            pl.BlockSpec((EDGE_IN, 2 * _EBLK), lambda i: (0, i)),
            pl.BlockSpec((2 * H, 4 * H), lambda i: (0, 0)),
            pl.BlockSpec((2 * H, 4 * H), lambda i: (0, 0)),
            pl.BlockSpec((EDGE_IN, 2 * H), lambda i: (0, 0)),
            pl.BlockSpec((1, 4 * H), lambda i: (0, 0)),
            pl.BlockSpec((4 * H, 2 * H), lambda i: (0, 0)),
            pl.BlockSpec((1, 2 * H), lambda i: (0, 0)),
        ],
        out_specs=pl.BlockSpec((_EBLK, 2 * H), lambda i: (i, 0)),
        out_shape=jax.ShapeDtypeStruct((_EP, 2 * H), f32),
        compiler_params=pltpu.CompilerParams(
            dimension_semantics=("parallel",)),
    )(xs2, xd2, efT, w1a, w1b, wec, b1, w2, b2)


def _upd_body(x, agg, u1a, u1b, b1, w2, b2, g, bb, o):
    t = jnp.dot(x[...], u1a[...], preferred_element_type=f32)
    t += jnp.dot(agg[...], u1b[...], preferred_element_type=f32)
    t = jnp.maximum(t + b1[...], 0.0)
    u = jnp.dot(t, w2[...], preferred_element_type=f32) + b2[...]
    y = x[...] + u
    ya = y[:, :H]
    yb = y[:, H:]

    def ln(z):
        m = jnp.mean(z, axis=-1, keepdims=True)
        c = z - m
        v = jnp.mean(c * c, axis=-1, keepdims=True)
        return c * lax.rsqrt(v + 1e-5)

    o[...] = jnp.concatenate([ln(ya), ln(yb)], axis=-1) * g[...] + bb[...]


def _upd_mlp(x2, agg2, u1a, u1b, b1, u2, b2, g, bb):
    return pl.pallas_call(
        _upd_body,
        grid=(_NGRID,),
        in_specs=[
            pl.BlockSpec((_NBLK, 2 * H), lambda i: (i, 0)),
            pl.BlockSpec((_NBLK, 2 * H), lambda i: (i, 0)),
            pl.BlockSpec((2 * H, 2 * H), lambda i: (0, 0)),
            pl.BlockSpec((2 * H, 2 * H), lambda i: (0, 0)),
            pl.BlockSpec((1, 2 * H), lambda i: (0, 0)),
            pl.BlockSpec((2 * H, 2 * H), lambda i: (0, 0)),
            pl.BlockSpec((1, 2 * H), lambda i: (0, 0)),
            pl.BlockSpec((1, 2 * H), lambda i: (0, 0)),
            pl.BlockSpec((1, 2 * H), lambda i: (0, 0)),
        ],
        out_specs=pl.BlockSpec((_NBLK, 2 * H), lambda i: (i, 0)),
        out_shape=jax.ShapeDtypeStruct((_NP, 2 * H), f32),
        compiler_params=pltpu.CompilerParams(
            dimension_semantics=("parallel",)),
    )(x2, agg2, u1a, u1b, b1, u2, b2, g, bb)


def _att_body(x, w1, b1, w2, b2, lo, mo, acc):
    i = pl.program_id(0)

    @pl.when(i == 0)
    def _():
        acc[...] = jnp.full((1, 1), -jnp.inf, f32)

    t = jnp.maximum(jnp.dot(x[...], w1[...], preferred_element_type=f32)
                    + b1[...], 0.0)
    l = jnp.dot(t, w2[...], preferred_element_type=f32) + b2[...]
    lo[...] = l
    acc[...] = jnp.maximum(acc[...], jnp.max(l, keepdims=True))

    @pl.when(i == _NGRID - 1)
    def _():
        mo[...] = acc[...]


def _att_logits(x2, w1, b1, w2, b2):
    return pl.pallas_call(
        _att_body,
        grid=(_NGRID,),
        in_specs=[
            pl.BlockSpec((_NBLK, 2 * H), lambda i: (i, 0)),
            pl.BlockSpec((2 * H, H), lambda i: (0, 0)),
            pl.BlockSpec((1, H), lambda i: (0, 0)),
            pl.BlockSpec((H, 2), lambda i: (0, 0)),
            pl.BlockSpec((1, 2), lambda i: (0, 0)),
        ],
        out_specs=[
            pl.BlockSpec((_NBLK, 2), lambda i: (i, 0)),
            pl.BlockSpec((1, 1), lambda i: (0, 0)),
        ],
        out_shape=[
            jax.ShapeDtypeStruct((_NP, 2), f32),
            jax.ShapeDtypeStruct((1, 1), f32),
        ],
        scratch_shapes=[pltpu.VMEM((1, 1), f32)],
        compiler_params=pltpu.CompilerParams(
            dimension_semantics=("arbitrary",)),
    )(x2, w1, b1, w2, b2)


def _pool_body(x, l, bidx, gmax, pw, pb, fw, fb, pred, feat, gf_acc, w_acc):
    i = pl.program_id(0)

    @pl.when(i == 0)
    def _():
        gf_acc[...] = jnp.zeros((B, H), f32)
        w_acc[...] = jnp.zeros((1, 1), f32)

    w = jnp.exp(l[...] - gmax[...])
    seg = bidx[0]
    iota = lax.broadcasted_iota(jnp.int32, (_NBLK, B), 1)
    oh_e = (seg[:, 0:1] == iota).astype(f32)
    oh_o = (seg[:, 1:2] == iota).astype(f32)
    xe = x[:, :H] * w[:, 0:1]
    xo = x[:, H:] * w[:, 1:2]
    dn = (((0,), (0,)), ((), ()))
    gf_acc[...] += (lax.dot_general(oh_e, xe, dn, preferred_element_type=f32)
                    + lax.dot_general(oh_o, xo, dn,
                                      preferred_element_type=f32))
    w_acc[...] += jnp.sum(w, keepdims=True).reshape(1, 1)

    @pl.when(i == _NGRID - 1)
    def _():
        gf = gf_acc[...] / w_acc[...]
        pred[...] = jnp.dot(gf, pw[...], preferred_element_type=f32) + pb[...]
        feat[...] = jnp.dot(gf, fw[...], preferred_element_type=f32) + fb[...]


def _pool(x2, l2, bidx3, gmax, pw, pb, fw, fb):
    return pl.pallas_call(
        _pool_body,
        grid=(_NGRID,),
        in_specs=[
            pl.BlockSpec((_NBLK, 2 * H), lambda i: (i, 0)),
            pl.BlockSpec((_NBLK, 2), lambda i: (i, 0)),
            pl.BlockSpec((1, _NBLK, 2), lambda i: (i, 0, 0)),
            pl.BlockSpec((1, 1), lambda i: (0, 0)),
            pl.BlockSpec((H, 1), lambda i: (0, 0)),
            pl.BlockSpec((1, 1), lambda i: (0, 0)),
            pl.BlockSpec((H, H), lambda i: (0, 0)),
            pl.BlockSpec((1, H), lambda i: (0, 0)),
        ],
        out_specs=[
            pl.BlockSpec((B, 1), lambda i: (0, 0)),
            pl.BlockSpec((B, H), lambda i: (0, 0)),
        ],
        out_shape=[
            jax.ShapeDtypeStruct((B, 1), f32),
            jax.ShapeDtypeStruct((B, H), f32),
        ],
        scratch_shapes=[pltpu.VMEM((B, H), f32), pltpu.VMEM((1, 1), f32)],
        compiler_params=pltpu.CompilerParams(
            dimension_semantics=("arbitrary",)),
    )(x2, l2, bidx3, gmax, pw, pb, fw, fb)


_CW = 128
_CH = E // _CW
_NWORK = 32
_SHARD = N // 2
_PAD = _SHARD + 24
_GS = 3
_GSUP = 195 // _GS


def _gather_body(x_hbm, s_hbm, d_hbm, xs_hbm, xd_hbm,
                 idx_s, idx_d, rs, rd, sem_i, sem_g, sem_w):
    xt = x_hbm
    xsf = xs_hbm
    xdf = xd_hbm
    c = lax.axis_index("c")
    s = lax.axis_index("s")
    w = s * 2 + c
    start = 195 * w + jnp.minimum(w, 10)

    def idx_cps(j, p):
        row = start + _GS * j
        return (pltpu.make_async_copy(s_hbm.at[pl.ds(row, _GS)],
                                      idx_s.at[p], sem_i),
                pltpu.make_async_copy(d_hbm.at[pl.ds(row, _GS)],
                                      idx_d.at[p], sem_i))

    def wb_cps(j, p):
        base = (start + _GS * j) * _CW
        return (pltpu.make_async_copy(rs.at[p],
                                      xsf.at[pl.ds(base, _GS * _CW)], sem_w),
                pltpu.make_async_copy(rd.at[p],
                                      xdf.at[pl.ds(base, _GS * _CW)], sem_w))

    for cp in idx_cps(0, 0):
        cp.start()

    def body(j, _):
        p = jnp.bitwise_and(j, 1)
        for cp in idx_cps(j, p):
            cp.wait()

        @pl.when(j < _GSUP - 1)
        def _():
            for cp in idx_cps(j + 1, 1 - p):
                cp.start()

        @pl.when(j >= 2)
        def _():
            for cp in wb_cps(j - 2, p):
                cp.wait()

        gcps = []
        for q in range(_GS):
            gcps.append(pltpu.make_async_copy(
                xt.at[idx_s.at[p, q]],
                rs.at[p, pl.ds(q * _CW, _CW)], sem_g))
            gcps.append(pltpu.make_async_copy(
                xt.at[idx_d.at[p, q]],
                rd.at[p, pl.ds(q * _CW, _CW)], sem_g))
        for cp in gcps:
            cp.start()
        for cp in gcps:
            cp.wait()
        for cp in wb_cps(j, p):
            cp.start()
        return 0

    lax.fori_loop(0, _GSUP, body, 0, unroll=False)
    for cp in wb_cps(_GSUP - 2, 1):
        cp.wait()
    for cp in wb_cps(_GSUP - 1, 0):
        cp.wait()

    @pl.when(w < 10)
    def _():
        row = start + 195
        pltpu.sync_copy(s_hbm.at[row], idx_s.at[0, 0])
        pltpu.sync_copy(d_hbm.at[row], idx_d.at[0, 0])
        cp1 = pltpu.async_copy(xt.at[idx_s.at[0, 0]],
                               rs.at[0, pl.ds(0, _CW)], sem_g)
        cp2 = pltpu.async_copy(xt.at[idx_d.at[0, 0]],
                               rd.at[0, pl.ds(0, _CW)], sem_g)
        cp1.wait()
        cp2.wait()
        pltpu.sync_copy(rs.at[0, pl.ds(0, _CW)],
                        xsf.at[pl.ds(row * _CW, _CW)])
        pltpu.sync_copy(rd.at[0, pl.ds(0, _CW)],
                        xdf.at[pl.ds(row * _CW, _CW)])


def _gather(x2, src2, dst2):
    k = pl.kernel(
        _gather_body,
        out_type=(jax.ShapeDtypeStruct((E, H), f32),
                  jax.ShapeDtypeStruct((E, H), f32)),
        mesh=plsc.VectorSubcoreMesh(core_axis_name="c", subcore_axis_name="s"),
        scratch_types=[
            pltpu.VMEM((2, _GS, _CW), jnp.int32),
            pltpu.VMEM((2, _GS, _CW), jnp.int32),
            pltpu.VMEM((2, _GS * _CW, H), f32),
            pltpu.VMEM((2, _GS * _CW, H), f32),
            pltpu.SemaphoreType.DMA,
            pltpu.SemaphoreType.DMA,
            pltpu.SemaphoreType.DMA,
        ],
        compiler_params=pltpu.CompilerParams(use_tc_tiling_on_sc=False),
    )
    return k(x2, src2, dst2)


def _scatter_body(m_hbm, d_hbm, z_hbm, agg_hbm, idx_v, idx_l, m_v, shard,
                  sem_i, sem_a):
    mf = m_hbm
    zf = z_hbm
    aggf = agg_hbm
    c = lax.axis_index("c")
    s = lax.axis_index("s")
    base = c * _SHARD
    zrows = _PAD // 16
    pltpu.sync_copy(zf.at[pl.ds(s * zrows, zrows)],
                    shard.at[pl.ds(s * zrows, zrows)])
    plsc.subcore_barrier()

    start = 390 * s + jnp.minimum(s, 10)
    n = jnp.where(s < 10, 391, 390)

    def pf_cps(k, p):
        r = start + k
        return (pltpu.make_async_copy(d_hbm.at[r], idx_v.at[p], sem_i),
                pltpu.make_async_copy(mf.at[pl.ds(r * _CW, _CW)],
                                      m_v.at[p], sem_i))

    def add_cp(p):
        return pltpu.make_async_copy(m_v.at[p], shard.at[idx_l.at[p]], sem_a)

    for cp in pf_cps(0, 0):
        cp.start()

    def body(k, _):
        p = jnp.remainder(k, 3)

        @pl.when(k < n)
        def _():
            for cp in pf_cps(k, p):
                cp.wait()
            for j in range(_CW // 16):
                v = idx_v[p, pl.ds(j * 16, 16)]
                inb = jnp.logical_and(v >= base, v < base + _SHARD)
                idx_l[p, pl.ds(j * 16, 16)] = jnp.where(inb, v - base,
                                                        _SHARD)
            pltpu.async_copy(m_v.at[p], shard.at[idx_l.at[p]], sem_a,
                             add=True)

            @pl.when(k >= 2)
            def _():
                add_cp(jnp.remainder(k - 2, 3)).wait()

            @pl.when(k + 1 < n)
            def _():
                for cp in pf_cps(k + 1, jnp.remainder(k + 1, 3)):
                    cp.start()
        return 0

    lax.fori_loop(0, 391, body, 0, unroll=False)
    add_cp(jnp.remainder(n - 2, 3)).wait()
    add_cp(jnp.remainder(n - 1, 3)).wait()
    plsc.subcore_barrier()

    wrows = _SHARD // 16
    r0 = s * wrows
    pltpu.sync_copy(shard.at[pl.ds(r0, wrows)],
                    aggf.at[pl.ds(base + r0, wrows)])

    @pl.when(s == 15)
    def _():
        pltpu.sync_copy(shard.at[pl.ds(16 * wrows, _SHARD - 16 * wrows)],
                        aggf.at[pl.ds(base + 16 * wrows,
                                      _SHARD - 16 * wrows)])


def _scatter(m2, dst2, zeros2):
    k = pl.kernel(
        _scatter_body,
        out_type=jax.ShapeDtypeStruct((N, H), f32),
        mesh=plsc.VectorSubcoreMesh(core_axis_name="c", subcore_axis_name="s"),
        scratch_types=[
            pltpu.VMEM((3, _CW), jnp.int32),
            pltpu.VMEM((3, _CW), jnp.int32),
            pltpu.VMEM((3, _CW, H), f32),
            pltpu.VMEM_SHARED((_PAD, H), f32),
            pltpu.SemaphoreType.DMA,
            pltpu.SemaphoreType.DMA,
        ],
        compiler_params=pltpu.CompilerParams(use_tc_tiling_on_sc=False),
    )
    return k(m2, dst2, zeros2)


def _bd(w):
    a, b = w.shape
    z = jnp.zeros((2 * a, 2 * b), f32)
    return z.at[:a, :b].set(w).at[a:, b:].set(w)


def _db(b):
    return jnp.concatenate([b, b])[None, :]


def _phi(v):
    i = v // (2 * _NBLK)
    j = v - i * (2 * _NBLK)
    return i * (2 * _NBLK) + (j % _NBLK) * 2 + j // _NBLK


def _eperm(a):
    return a.reshape(_EGRID, 2, _EBLK).transpose(0, 2, 1).reshape(_CH, _CW)


def kernel(node_features, edge_features, edge_index, batch_index, node_W,
           node_b, edge_W, edge_b, msg_W1, msg_b1, msg_W2, msg_b2, upd_W1,
           upd_b1, upd_W2, upd_b2, ln_g, ln_b, att_W1, att_b1, att_W2,
           att_b2, prop_W, prop_b, feat_W, feat_b):
    src2 = _eperm(_phi(edge_index[0]))
    dst2 = _eperm(_phi(edge_index[1]))
    x2 = _proj_nodes(node_features, node_W, node_b.reshape(1, H))
    efT = edge_features.T
    zeros = jnp.zeros((_PAD, H), f32)
    for l in range(3):
        xs, xd = _gather(x2.reshape(N, H), src2, dst2)
        w1 = msg_W1[l]
        wec = edge_W @ w1[2 * H:]
        bec = edge_b @ w1[2 * H:] + msg_b1[l]
        m2 = _msg_mlp(xs.reshape(_EP, 2 * H), xd.reshape(_EP, 2 * H), efT,
                      _bd(w1[:H]), _bd(w1[H:2 * H]),
                      wec, _db(bec), _bd(msg_W2[l]),
                      _db(msg_b2[l]))
        agg2 = _scatter(m2.reshape(E, H), dst2, zeros).reshape(_NP, 2 * H)
        x2 = _upd_mlp(x2, agg2, _bd(upd_W1[l][:H]), _bd(upd_W1[l][H:]),
                      _db(upd_b1[l]), _bd(upd_W2[l]), _db(upd_b2[l]),
                      _db(ln_g[l]), _db(ln_b[l]))
    l2, gmax = _att_logits(x2, _bd(att_W1), _db(att_b1), _bd(att_W2),
                           _db(att_b2))
    bidx3 = batch_index.reshape(_NGRID, 2, _NBLK).transpose(0, 2, 1)
    pred, feat = _pool(x2, l2, bidx3, gmax, prop_W, prop_b.reshape(1, 1),
                       feat_W, feat_b.reshape(1, H))
    xout = (x2.reshape(_NGRID, _NBLK, 2, H).transpose(0, 2, 1, 3)
            .reshape(N, H))
    return (pred, feat, xout)
